# baseline (speedup 1.0000x reference)
import functools

import jax
import jax.numpy as jnp
from jax import lax
from jax.experimental import pallas as pl
from jax.experimental.pallas import tpu as pltpu
from jax.experimental.pallas import tpu_sc as plsc

B, N, M, C, L, K = 16, 16384, 8192, 256, 3, 4096
LN = 16
NC, NS = 2, 16
NW = NC * NS
ROWS_PER_W = (B * N) // NW
CHUNK = 64
NCHUNKS = ROWS_PER_W // CHUNK


def _wid():
    return lax.axis_index("s") * NC + lax.axis_index("c")


def _index_kernel_body(mask_hbm, vcs_hbm, vcd_hbm, gidx_hbm,
                       mask_v, vsrc_v, vdst_v, g0_v, g1_v, g2_v, inv_v, obuf_v):
    wid = _wid()

    @pl.when(wid < B)
    def _():
        b = wid
        lanes = lax.iota(jnp.int32, LN)

        pltpu.sync_copy(mask_hbm.at[pl.ds(b * M, M)], mask_v)

        neg1 = jnp.full((LN,), -1, jnp.int32)

        def init_inv(ci, carry):
            inv_v[pl.ds(ci * LN, LN)] = neg1
            return carry
        lax.fori_loop(0, N // LN, init_inv, 0)

        def scat_inv(ci, carry):
            keys = mask_v[pl.ds(ci * LN, LN)]
            mvals = ci * LN + lanes
            for l in range(LN):
                plsc.store_scatter(inv_v, [keys], mvals, mask=lanes == l)
            return carry
        lax.fori_loop(0, M // LN, scat_inv, 0)

        for lvl, g_ref in ((0, g0_v), (1, g1_v), (2, g2_v)):
            pltpu.sync_copy(vcs_hbm.at[pl.ds(lvl * K, K)], vsrc_v)
            pltpu.sync_copy(vcd_hbm.at[pl.ds(lvl * K, K)], vdst_v)

            def init_g(ci, carry, g_ref=g_ref):
                g_ref[pl.ds(ci * LN, LN)] = ci * LN + lanes
                return carry
            lax.fori_loop(0, N // LN, init_g, 0)

            def scat_g(ci, carry, g_ref=g_ref):
                dst = vdst_v[pl.ds(ci * LN, LN)]
                src = vsrc_v[pl.ds(ci * LN, LN)]
                for l in range(LN):
                    plsc.store_scatter(g_ref, [dst], src, mask=lanes == l)
                return carry
            lax.fori_loop(0, K // LN, scat_g, 0)

        def compose(ci, carry):
            a = g0_v[pl.ds(ci * LN, LN)]
            t = plsc.load_gather(g1_v, [a])
            t = plsc.load_gather(g2_v, [t])
            s = plsc.load_gather(inv_v, [t])
            obuf_v[pl.ds(ci * LN, LN)] = jnp.where(s >= 0, b * M + s, -1)
            return carry
        lax.fori_loop(0, N // LN, compose, 0)

        pltpu.sync_copy(obuf_v, gidx_hbm.at[pl.ds(b * N, N)])


NBUF = 4


def _gather_kernel_body(in_hbm, gidx_hbm, out_hbm, *scratch):
    raws = scratch[0:NBUF]
    idxs = scratch[NBUF:2 * NBUF]
    scales = scratch[2 * NBUF:3 * NBUF]
    rowss = scratch[3 * NBUF:4 * NBUF]
    sems = scratch[4 * NBUF:5 * NBUF]
    wid = _wid()
    start = wid * ROWS_PER_W

    def fire(c, b):
        base = start + c * CHUNK
        pltpu.sync_copy(gidx_hbm.at[pl.ds(base, CHUNK)], raws[b])
        for j in range(CHUNK // LN):
            g = raws[b][pl.ds(j * LN, LN)]
            idxs[b][pl.ds(j * LN, LN)] = jnp.maximum(g, 0)
            scales[b][pl.ds(j * LN, LN)] = jnp.where(
                g >= 0, jnp.float32(1.0), jnp.float32(0.0))
        pltpu.async_copy(in_hbm.at[idxs[b]], rowss[b], sems[b])

    def drain_store(c, b):
        pltpu.make_async_copy(in_hbm.at[idxs[b]], rowss[b], sems[b]).wait()

        def row_body(r, rcarry):
            bvec = plsc.load_gather(scales[b], [jnp.full((LN,), 0, jnp.int32) + r])
            for h in range(C // 128):
                for j in range(128 // LN):
                    rowss[b][r, h, pl.ds(j * LN, LN)] = (
                        rowss[b][r, h, pl.ds(j * LN, LN)] * bvec)
            return rcarry
        lax.fori_loop(0, CHUNK, row_body, 0)

        pltpu.sync_copy(rowss[b], out_hbm.at[pl.ds(start + c * CHUNK, CHUNK)])

    for b in range(NBUF):
        fire(b, b)

    def outer(it, carry):
        for b in range(NBUF):
            c = it * NBUF + b
            drain_store(c, b)
            nxt = c + NBUF

            @pl.when(nxt < NCHUNKS)
            def _(nxt=nxt, b=b):
                fire(nxt, b)
        return carry

    lax.fori_loop(0, NCHUNKS // NBUF, outer, 0)


def _build_calls():
    mesh = plsc.VectorSubcoreMesh(core_axis_name="c", subcore_axis_name="s")
    params = pltpu.CompilerParams(needs_layout_passes=False)

    index_call = functools.partial(
        pl.kernel,
        mesh=mesh,
        compiler_params=params,
        out_type=jax.ShapeDtypeStruct((B * N,), jnp.int32),
        scratch_types=[
            pltpu.VMEM((M,), jnp.int32),
            pltpu.VMEM((K,), jnp.int32),
            pltpu.VMEM((K,), jnp.int32),
            pltpu.VMEM((N,), jnp.int32),
            pltpu.VMEM((N,), jnp.int32),
            pltpu.VMEM((N,), jnp.int32),
            pltpu.VMEM((N,), jnp.int32),
            pltpu.VMEM((N,), jnp.int32),
        ],
    )(_index_kernel_body)

    gather_call = functools.partial(
        pl.kernel,
        mesh=mesh,
        compiler_params=params,
        out_type=jax.ShapeDtypeStruct((B * N, C // 128, 128), jnp.float32),
        scratch_types=(
            [pltpu.VMEM((CHUNK,), jnp.int32) for _ in range(NBUF)]
            + [pltpu.VMEM((CHUNK,), jnp.int32) for _ in range(NBUF)]
            + [pltpu.VMEM((CHUNK,), jnp.float32) for _ in range(NBUF)]
            + [pltpu.VMEM((CHUNK, C // 128, 128), jnp.float32) for _ in range(NBUF)]
            + [pltpu.SemaphoreType.DMA for _ in range(NBUF)]
        ),
    )(_gather_kernel_body)

    return index_call, gather_call


_INDEX_CALL, _GATHER_CALL = _build_calls()


def kernel(out, mask, in_images, vc_order):
    assert out.shape == (B, N, C) and mask.shape == (B, M)
    assert in_images.shape == (B, M, C) and vc_order.shape == (L, K, 2)
    vc_src = vc_order[:, :, 0].reshape(L * K)
    vc_dst = vc_order[:, :, 1].reshape(L * K)
    gidx = _INDEX_CALL(mask.reshape(B * M), vc_src, vc_dst)
    out_img = _GATHER_CALL(in_images.reshape(B * M, C // 128, 128), gidx)
    return out_img.reshape(B, N, C)


# speedup vs baseline: 1.0612x; 1.0612x over previous
import functools

import jax
import jax.numpy as jnp
from jax import lax
from jax.experimental import pallas as pl
from jax.experimental.pallas import tpu as pltpu
from jax.experimental.pallas import tpu_sc as plsc

B, N, M, C, L, K = 16, 16384, 8192, 256, 3, 4096
LN = 16
NC, NS = 2, 16
NW = NC * NS
ROWS_PER_W = (B * N) // NW
CHUNK = 64
NCHUNKS = ROWS_PER_W // CHUNK


def _wid():
    return lax.axis_index("s") * NC + lax.axis_index("c")


def _index_kernel_body(mask_hbm, vcs_hbm, vcd_hbm, gidx_hbm,
                       mask_v, vsrc_v, vdst_v, g0_v, g1_v, g2_v, inv_v, obuf_v):
    wid = _wid()

    @pl.when(wid < B)
    def _():
        b = wid
        lanes = lax.iota(jnp.int32, LN)

        pltpu.sync_copy(mask_hbm.at[pl.ds(b * M, M)], mask_v)

        neg1 = jnp.full((LN,), -1, jnp.int32)

        def init_inv(ci, carry):
            inv_v[pl.ds(ci * LN, LN)] = neg1
            return carry
        lax.fori_loop(0, N // LN, init_inv, 0)

        def scat_inv(ci, carry):
            keys = mask_v[pl.ds(ci * LN, LN)]
            mvals = ci * LN + lanes
            for l in range(LN):
                plsc.store_scatter(inv_v, [keys], mvals, mask=lanes == l)
            return carry
        lax.fori_loop(0, M // LN, scat_inv, 0)

        for lvl, g_ref in ((0, g0_v), (1, g1_v), (2, g2_v)):
            pltpu.sync_copy(vcs_hbm.at[pl.ds(lvl * K, K)], vsrc_v)
            pltpu.sync_copy(vcd_hbm.at[pl.ds(lvl * K, K)], vdst_v)

            def init_g(ci, carry, g_ref=g_ref):
                g_ref[pl.ds(ci * LN, LN)] = ci * LN + lanes
                return carry
            lax.fori_loop(0, N // LN, init_g, 0)

            def scat_g(ci, carry, g_ref=g_ref):
                dst = vdst_v[pl.ds(ci * LN, LN)]
                src = vsrc_v[pl.ds(ci * LN, LN)]
                for l in range(LN):
                    plsc.store_scatter(g_ref, [dst], src, mask=lanes == l)
                return carry
            lax.fori_loop(0, K // LN, scat_g, 0)

        def compose(ci, carry):
            a = g0_v[pl.ds(ci * LN, LN)]
            t = plsc.load_gather(g1_v, [a])
            t = plsc.load_gather(g2_v, [t])
            s = plsc.load_gather(inv_v, [t])
            obuf_v[pl.ds(ci * LN, LN)] = jnp.where(s >= 0, b * M + s, -1)
            return carry
        lax.fori_loop(0, N // LN, compose, 0)

        pltpu.sync_copy(obuf_v, gidx_hbm.at[pl.ds(b * N, N)])


NBUF = 4


def _gather_kernel_body(in_hbm, gidx_hbm, out_hbm, *scratch):
    raws = scratch[0:NBUF]
    raws_s = scratch[NBUF:2 * NBUF]
    scales = scratch[2 * NBUF:3 * NBUF]
    rowss = scratch[3 * NBUF:4 * NBUF]
    sems = scratch[4 * NBUF:5 * NBUF]
    wid = _wid()
    start = wid * ROWS_PER_W

    def fire(c, b):
        base = start + c * CHUNK
        lanes = lax.iota(jnp.int32, LN)
        pltpu.sync_copy(gidx_hbm.at[pl.ds(base, CHUNK)], raws[b])

        def issue_grp(j, rcarry):
            gv = jnp.maximum(raws[b][pl.ds(j * LN, LN)], 0)
            scales[b][pl.ds(j * LN, LN)] = jnp.where(
                raws[b][pl.ds(j * LN, LN)] >= 0, jnp.float32(1.0), jnp.float32(0.0))
            for l in range(LN):
                src = jnp.max(jnp.where(lanes == l, gv, 0))
                pltpu.async_copy(in_hbm.at[pl.ds(src, 1)],
                                 rowss[b].at[pl.ds(j * LN + l, 1)], sems[b])
            return rcarry
        lax.fori_loop(0, CHUNK // LN, issue_grp, 0)

    def drain_store(c, b):
        pltpu.make_async_copy(in_hbm.at[pl.ds(0, CHUNK)], rowss[b], sems[b]).wait()

        def row_body(r, rcarry):
            bvec = plsc.load_gather(scales[b], [jnp.full((LN,), 0, jnp.int32) + r])
            for j in range(C // LN):
                rowss[b][r, pl.ds(j * LN, LN)] = rowss[b][r, pl.ds(j * LN, LN)] * bvec
            return rcarry
        lax.fori_loop(0, CHUNK, row_body, 0)

        pltpu.sync_copy(rowss[b], out_hbm.at[pl.ds(start + c * CHUNK, CHUNK)])

    for b in range(NBUF):
        fire(b, b)

    def outer(it, carry):
        for b in range(NBUF):
            c = it * NBUF + b
            drain_store(c, b)
            nxt = c + NBUF

            @pl.when(nxt < NCHUNKS)
            def _(nxt=nxt, b=b):
                fire(nxt, b)
        return carry

    lax.fori_loop(0, NCHUNKS // NBUF, outer, 0)


def _build_calls():
    mesh = plsc.VectorSubcoreMesh(core_axis_name="c", subcore_axis_name="s")
    params = pltpu.CompilerParams(needs_layout_passes=False)

    index_call = functools.partial(
        pl.kernel,
        mesh=mesh,
        compiler_params=params,
        out_type=jax.ShapeDtypeStruct((B * N,), jnp.int32),
        scratch_types=[
            pltpu.VMEM((M,), jnp.int32),
            pltpu.VMEM((K,), jnp.int32),
            pltpu.VMEM((K,), jnp.int32),
            pltpu.VMEM((N,), jnp.int32),
            pltpu.VMEM((N,), jnp.int32),
            pltpu.VMEM((N,), jnp.int32),
            pltpu.VMEM((N,), jnp.int32),
            pltpu.VMEM((N,), jnp.int32),
        ],
    )(_index_kernel_body)

    gather_call = functools.partial(
        pl.kernel,
        mesh=mesh,
        compiler_params=params,
        out_type=jax.ShapeDtypeStruct((B * N, C), jnp.float32),
        scratch_types=(
            [pltpu.VMEM((CHUNK,), jnp.int32) for _ in range(NBUF)]
            + [pltpu.SMEM((CHUNK,), jnp.int32) for _ in range(NBUF)]
            + [pltpu.VMEM((CHUNK,), jnp.float32) for _ in range(NBUF)]
            + [pltpu.VMEM((CHUNK, C), jnp.float32) for _ in range(NBUF)]
            + [pltpu.SemaphoreType.DMA for _ in range(NBUF)]
        ),
    )(_gather_kernel_body)

    return index_call, gather_call


_INDEX_CALL, _GATHER_CALL = _build_calls()


def kernel(out, mask, in_images, vc_order):
    assert out.shape == (B, N, C) and mask.shape == (B, M)
    assert in_images.shape == (B, M, C) and vc_order.shape == (L, K, 2)
    vc_src = vc_order[:, :, 0].reshape(L * K)
    vc_dst = vc_order[:, :, 1].reshape(L * K)
    gidx = _INDEX_CALL(mask.reshape(B * M), vc_src, vc_dst)
    out_img = _GATHER_CALL(in_images.reshape(B * M, C), gidx)
    return out_img.reshape(B, N, C)


# speedup vs baseline: 17.4759x; 16.4674x over previous
import functools

import jax
import jax.numpy as jnp
from jax import lax
from jax.experimental import pallas as pl
from jax.experimental.pallas import tpu as pltpu
from jax.experimental.pallas import tpu_sc as plsc

B, N, M, C, L, K = 16, 16384, 8192, 256, 3, 4096
LN = 16
NC, NS = 2, 16
NW = NC * NS
ROWS_PER_W = (B * N) // NW
CHUNK = 64
NCHUNKS = ROWS_PER_W // CHUNK


def _wid():
    return lax.axis_index("s") * NC + lax.axis_index("c")


def _index_kernel_body(mask_hbm, vcs_hbm, vcd_hbm, gidx_hbm,
                       mask_v, vsrc_v, vdst_v, g0_v, g1_v, g2_v, inv_v, obuf_v):
    wid = _wid()

    @pl.when(wid < B)
    def _():
        b = wid
        lanes = lax.iota(jnp.int32, LN)

        pltpu.sync_copy(mask_hbm.at[pl.ds(b * M, M)], mask_v)

        neg1 = jnp.full((LN,), -1, jnp.int32)

        def init_inv(ci, carry):
            inv_v[pl.ds(ci * LN, LN)] = neg1
            return carry
        lax.fori_loop(0, N // LN, init_inv, 0)

        def scat_inv(ci, carry):
            keys = mask_v[pl.ds(ci * LN, LN)]
            mvals = ci * LN + lanes
            for l in range(LN):
                plsc.store_scatter(inv_v, [keys], mvals, mask=lanes == l)
            return carry
        lax.fori_loop(0, M // LN, scat_inv, 0)

        for lvl, g_ref in ((0, g0_v), (1, g1_v), (2, g2_v)):
            pltpu.sync_copy(vcs_hbm.at[pl.ds(lvl * K, K)], vsrc_v)
            pltpu.sync_copy(vcd_hbm.at[pl.ds(lvl * K, K)], vdst_v)

            def init_g(ci, carry, g_ref=g_ref):
                g_ref[pl.ds(ci * LN, LN)] = ci * LN + lanes
                return carry
            lax.fori_loop(0, N // LN, init_g, 0)

            def scat_g(ci, carry, g_ref=g_ref):
                dst = vdst_v[pl.ds(ci * LN, LN)]
                src = vsrc_v[pl.ds(ci * LN, LN)]
                for l in range(LN):
                    plsc.store_scatter(g_ref, [dst], src, mask=lanes == l)
                return carry
            lax.fori_loop(0, K // LN, scat_g, 0)

        def compose(ci, carry):
            a = g0_v[pl.ds(ci * LN, LN)]
            t = plsc.load_gather(g1_v, [a])
            t = plsc.load_gather(g2_v, [t])
            s = plsc.load_gather(inv_v, [t])
            obuf_v[pl.ds(ci * LN, LN)] = jnp.where(s >= 0, b * M + s, -1)
            return carry
        lax.fori_loop(0, N // LN, compose, 0)

        pltpu.sync_copy(obuf_v, gidx_hbm.at[pl.ds(b * N, N)])


NBUF = 4


def _gather_kernel_body(in_hbm, gidx_hbm, out_hbm, *scratch):
    raws = scratch[0:NBUF]
    counts = scratch[NBUF:2 * NBUF]
    scales = scratch[2 * NBUF:3 * NBUF]
    rowss = scratch[3 * NBUF:4 * NBUF]
    sems = scratch[4 * NBUF:5 * NBUF]
    wid = _wid()
    start = wid * ROWS_PER_W

    zero_row = jnp.zeros((LN,), jnp.float32)
    for b in range(NBUF):
        def zinit(r, zcarry, b=b):
            for j in range(C // LN):
                rowss[b][r, pl.ds(j * LN, LN)] = zero_row
            return zcarry
        lax.fori_loop(0, CHUNK, zinit, 0)

    def fire(c, b):
        base = start + c * CHUNK
        lanes = lax.iota(jnp.int32, LN)
        pltpu.sync_copy(gidx_hbm.at[pl.ds(base, CHUNK)], raws[b])

        def issue_grp(j, ncarry):
            gv = raws[b][pl.ds(j * LN, LN)]
            valid = gv >= 0
            scales[b][pl.ds(j * LN, LN)] = jnp.where(
                valid, jnp.float32(1.0), jnp.float32(0.0))
            for l in range(LN):
                sraw = jnp.max(jnp.where(lanes == l, gv, jnp.int32(-1)))

                @pl.when(sraw >= 0)
                def _(sraw=sraw, l=l, j=j, b=b):
                    pltpu.async_copy(in_hbm.at[pl.ds(sraw, 1)],
                                     rowss[b].at[pl.ds(j * LN + l, 1)], sems[b])
            return ncarry + jnp.sum(jnp.where(valid, 1, 0))
        nvalid = lax.fori_loop(0, CHUNK // LN, issue_grp, jnp.int32(0))
        counts[b][0] = nvalid

    def drain_store(c, b):
        def drain_one(i, dcarry):
            pltpu.make_async_copy(in_hbm.at[pl.ds(0, 1)],
                                  rowss[b].at[pl.ds(0, 1)], sems[b]).wait()
            return dcarry
        lax.fori_loop(0, counts[b][0], drain_one, 0)

        def row_body(r, rcarry):
            bvec = plsc.load_gather(scales[b], [jnp.full((LN,), 0, jnp.int32) + r])
            for j in range(C // LN):
                rowss[b][r, pl.ds(j * LN, LN)] = rowss[b][r, pl.ds(j * LN, LN)] * bvec
            return rcarry
        lax.fori_loop(0, CHUNK, row_body, 0)

        pltpu.sync_copy(rowss[b], out_hbm.at[pl.ds(start + c * CHUNK, CHUNK)])

    for b in range(NBUF):
        fire(b, b)

    def outer(it, carry):
        for b in range(NBUF):
            c = it * NBUF + b
            drain_store(c, b)
            nxt = c + NBUF

            @pl.when(nxt < NCHUNKS)
            def _(nxt=nxt, b=b):
                fire(nxt, b)
        return carry

    lax.fori_loop(0, NCHUNKS // NBUF, outer, 0)


def _build_calls():
    mesh = plsc.VectorSubcoreMesh(core_axis_name="c", subcore_axis_name="s")
    params = pltpu.CompilerParams(needs_layout_passes=False)

    index_call = functools.partial(
        pl.kernel,
        mesh=mesh,
        compiler_params=params,
        out_type=jax.ShapeDtypeStruct((B * N,), jnp.int32),
        scratch_types=[
            pltpu.VMEM((M,), jnp.int32),
            pltpu.VMEM((K,), jnp.int32),
            pltpu.VMEM((K,), jnp.int32),
            pltpu.VMEM((N,), jnp.int32),
            pltpu.VMEM((N,), jnp.int32),
            pltpu.VMEM((N,), jnp.int32),
            pltpu.VMEM((N,), jnp.int32),
            pltpu.VMEM((N,), jnp.int32),
        ],
    )(_index_kernel_body)

    gather_call = functools.partial(
        pl.kernel,
        mesh=mesh,
        compiler_params=params,
        out_type=jax.ShapeDtypeStruct((B * N, C), jnp.float32),
        scratch_types=(
            [pltpu.VMEM((CHUNK,), jnp.int32) for _ in range(NBUF)]
            + [pltpu.SMEM((1,), jnp.int32) for _ in range(NBUF)]
            + [pltpu.VMEM((CHUNK,), jnp.float32) for _ in range(NBUF)]
            + [pltpu.VMEM((CHUNK, C), jnp.float32) for _ in range(NBUF)]
            + [pltpu.SemaphoreType.DMA for _ in range(NBUF)]
        ),
    )(_gather_kernel_body)

    return index_call, gather_call


_INDEX_CALL, _GATHER_CALL = _build_calls()


def kernel(out, mask, in_images, vc_order):
    assert out.shape == (B, N, C) and mask.shape == (B, M)
    assert in_images.shape == (B, M, C) and vc_order.shape == (L, K, 2)
    vc_src = vc_order[:, :, 0].reshape(L * K)
    vc_dst = vc_order[:, :, 1].reshape(L * K)
    gidx = _INDEX_CALL(mask.reshape(B * M), vc_src, vc_dst)
    out_img = _GATHER_CALL(in_images.reshape(B * M, C), gidx)
    return out_img.reshape(B, N, C)
